# baseline (speedup 1.0000x reference)
import functools

import jax
import jax.numpy as jnp
from jax import lax
from jax.experimental import pallas as pl
from jax.experimental.pallas import tpu as pltpu
from jax.experimental.pallas import tpu_sc as plsc

N_NODES = 10000
N_EDGES = 320000
D_FEAT = 128

NC = 2
NS = 16
LANES = 16
HFEAT = D_FEAT // NC
E_PER_T = N_EDGES // NS
BATCH = 80
NBUF = 5
CHUNK_B = 25
CHUNK = CHUNK_B * BATCH
NCHUNK = E_PER_T // CHUNK
N_PAD = 10240
ROWS_PER_TILE = N_PAD // NS
NSEG = HFEAT // LANES
EPB16 = BATCH // LANES


def _sc_spmm(adj_hbm, vals_hbm, embeds_hbm,
             out,
             g0, g1, g2, g3, g4,
             colv, rowv, valv, acc, etab,
             gs0, gs1, gs2, gs3, gs4, ss0, ss1, ss2, ss3, ss4,
             pc, pr, pv, pe):
    g = [g0, g1, g2, g3, g4]
    gsem = [gs0, gs1, gs2, gs3, gs4]
    ssem = [ss0, ss1, ss2, ss3, ss4]

    h = lax.axis_index("c")
    s = lax.axis_index("s")
    base = s * E_PER_T
    csl = pl.ds(h * HFEAT, HFEAT)

    fr0 = s * ROWS_PER_TILE
    tail_rows = N_NODES - (NS - 1) * ROWS_PER_TILE

    def fill_copy(nrows):
        return pltpu.make_async_copy(
            embeds_hbm.at[pl.ds(fr0, nrows), csl],
            etab.at[pl.ds(fr0, nrows)], pe)

    @pl.when(s < NS - 1)
    def _():
        fill_copy(ROWS_PER_TILE).start()

    @pl.when(s == NS - 1)
    def _():
        fill_copy(tail_rows).start()

    def chunk_copies(ch, p):
        off = base + ch * CHUNK
        dst = pl.ds(p * CHUNK, CHUNK)
        return (
            pltpu.make_async_copy(
                adj_hbm.at[1, pl.ds(off, CHUNK)], colv.at[dst], pc),
            pltpu.make_async_copy(
                adj_hbm.at[0, pl.ds(off, CHUNK)], rowv.at[dst], pr),
            pltpu.make_async_copy(
                vals_hbm.at[pl.ds(off, CHUNK)], valv.at[dst], pv),
        )

    def fire_chunk(ch, p):
        for cp in chunk_copies(ch, p):
            cp.start()

    def wait_chunk(ch, p):
        for cp in chunk_copies(ch, p):
            cp.wait()

    fire_chunk(0, 0)

    zeros16 = jnp.zeros((LANES,), jnp.float32)

    def zero_body(i, _):
        for k in range(NSEG):
            g4[i, pl.ds(k * LANES, LANES)] = zeros16
        return 0

    lax.fori_loop(0, BATCH, zero_body, 0)
    for m in range(ROWS_PER_TILE // BATCH):
        pltpu.sync_copy(g4, acc.at[pl.ds(s * ROWS_PER_TILE + m * BATCH, BATCH)])

    @pl.when(s < NS - 1)
    def _():
        fill_copy(ROWS_PER_TILE).wait()

    @pl.when(s == NS - 1)
    def _():
        fill_copy(tail_rows).wait()

    plsc.subcore_barrier()

    def gather_copy(p, b, j):
        idx = colv.at[pl.ds(p * CHUNK + b * BATCH, BATCH)]
        return pltpu.make_async_copy(etab.at[idx], g[j], gsem[j])

    def scatter_copy(p, b, j):
        idx = rowv.at[pl.ds(p * CHUNK + b * BATCH, BATCH)]
        return pltpu.make_async_copy(g[j], acc.at[idx], ssem[j])

    def scale(gj, p, b):
        def sb(j16, _):
            off = p * CHUNK + b * BATCH + j16 * LANES
            v16 = valv[pl.ds(off, LANES)]
            ebase = j16 * LANES
            for i in range(LANES):
                vv = jnp.full((LANES,), v16[i], jnp.float32)
                for k in range(NSEG):
                    sl = pl.ds(k * LANES, LANES)
                    gj[ebase + i, sl] = gj[ebase + i, sl] * vv
            return 0

        lax.fori_loop(0, EPB16, sb, 0)

    def chunk_body(ch, _):
        p = lax.rem(ch, 2)
        wait_chunk(ch, p)

        @pl.when(ch + 1 < NCHUNK)
        def _():
            fire_chunk(ch + 1, 1 - p)

        for j in range(NBUF - 1):
            gather_copy(p, j, j).start()

        def outer(a, _):
            for j in range(NBUF):
                b = a * NBUF + j
                j4 = (j + 4) % NBUF
                gather_copy(p, b, j).wait()
                scale(g[j], p, b)
                idx = rowv.at[pl.ds(p * CHUNK + b * BATCH, BATCH)]
                pltpu.async_copy(g[j], acc.at[idx], ssem[j], add=True)

                @pl.when(b >= 1)
                def _():
                    scatter_copy(p, b - 1, j4).wait()

                @pl.when(b + NBUF - 1 < CHUNK_B)
                def _():
                    gather_copy(p, b + NBUF - 1, j4).start()
            return 0

        lax.fori_loop(0, CHUNK_B // NBUF, outer, 0)
        scatter_copy(p, CHUNK_B - 1, (CHUNK_B - 1) % NBUF).wait()
        return 0

    lax.fori_loop(0, NCHUNK, chunk_body, 0)
    plsc.subcore_barrier()

    @pl.when(s < NS - 1)
    def _():
        rsl = pl.ds(s * ROWS_PER_TILE, ROWS_PER_TILE)
        pltpu.sync_copy(acc.at[rsl], out.at[rsl, csl])

    @pl.when(s == NS - 1)
    def _():
        rslt = pl.ds((NS - 1) * ROWS_PER_TILE, tail_rows)
        pltpu.sync_copy(acc.at[rslt], out.at[rslt, csl])


_sc_spmm_call = functools.partial(
    pl.kernel,
    out_type=jax.ShapeDtypeStruct((N_NODES, D_FEAT), jnp.float32),
    mesh=plsc.VectorSubcoreMesh(core_axis_name="c", subcore_axis_name="s"),
    compiler_params=pltpu.CompilerParams(use_tc_tiling_on_sc=False),
    scratch_types=(
        [pltpu.VMEM((BATCH, HFEAT), jnp.float32)] * NBUF
        + [
            pltpu.VMEM((2 * CHUNK,), jnp.int32),
            pltpu.VMEM((2 * CHUNK,), jnp.int32),
            pltpu.VMEM((2 * CHUNK,), jnp.float32),
            pltpu.VMEM_SHARED((N_PAD, HFEAT), jnp.float32),
            pltpu.VMEM_SHARED((N_PAD, HFEAT), jnp.float32),
        ]
        + [pltpu.SemaphoreType.DMA] * (2 * NBUF + 4)
    ),
)(_sc_spmm)


@jax.jit
def kernel(adj_indices, adj_values, embeds):
    adj = adj_indices.astype(jnp.int32)
    return _sc_spmm_call(adj, adj_values, embeds)


# speedup vs baseline: 2.7751x; 2.7751x over previous
import functools

import jax
import jax.numpy as jnp
from jax import lax
from jax.experimental import pallas as pl
from jax.experimental.pallas import tpu as pltpu
from jax.experimental.pallas import tpu_sc as plsc

N_NODES = 10000
N_EDGES = 320000
D_FEAT = 128

NC = 2
NS = 16
LANES = 16
HFEAT = D_FEAT // NC
E_PER_T = N_EDGES // NS
BATCH = 80
NBATCH = E_PER_T // BATCH
NBUF = 5
N_PAD = 10240
ROWS_PER_TILE = N_PAD // NS
NSEG = HFEAT // LANES
EPB16 = BATCH // LANES


def _sc_spmm(adj_hbm, vals_hbm, embeds_hbm,
             out,
             g0, g1, g2, g3, g4, r0, r1, r2, r3, r4,
             c0, c1, c2, c3, c4,
             colv, rowv, valv, acc,
             gs0, gs1, gs2, gs3, gs4, ss0, ss1, ss2, ss3, ss4,
             pc, pr, pv):
    g = [g0, g1, g2, g3, g4]
    r = [r0, r1, r2, r3, r4]
    cb = [c0, c1, c2, c3, c4]
    gsem = [gs0, gs1, gs2, gs3, gs4]
    ssem = [ss0, ss1, ss2, ss3, ss4]

    h = lax.axis_index("c")
    s = lax.axis_index("s")
    base = s * E_PER_T

    pltpu.async_copy(adj_hbm.at[1, pl.ds(base, E_PER_T)], colv, pc)
    pltpu.async_copy(adj_hbm.at[0, pl.ds(base, E_PER_T)], rowv, pr)
    pltpu.async_copy(vals_hbm.at[pl.ds(base, E_PER_T)], valv, pv)

    hvec = jnp.full((LANES,), h, jnp.int32)
    two = jnp.full((LANES,), 2, jnp.int32)

    def fire_gather(b, j):
        for i in range(EPB16):
            cb[j][pl.ds(i * LANES, LANES)] = \
                colv[pl.ds(b * BATCH + i * LANES, LANES)] * two + hvec
        pltpu.async_copy(embeds_hbm.at[cb[j]], g[j], gsem[j])

    def wait_gather(b, j):
        pltpu.make_async_copy(embeds_hbm.at[cb[j]], g[j], gsem[j]).wait()

    def wait_scatter(j):
        pltpu.make_async_copy(g[j], acc.at[r[j]], ssem[j]).wait()

    pltpu.make_async_copy(adj_hbm.at[1, pl.ds(base, E_PER_T)], colv, pc).wait()
    for j in range(NBUF - 1):
        fire_gather(j, j)

    zeros16 = jnp.zeros((LANES,), jnp.float32)

    def zero_body(i, _):
        for k in range(NSEG):
            g4[i, pl.ds(k * LANES, LANES)] = zeros16
        return 0

    lax.fori_loop(0, BATCH, zero_body, 0)
    for m in range(ROWS_PER_TILE // BATCH):
        pltpu.sync_copy(g4, acc.at[pl.ds(s * ROWS_PER_TILE + m * BATCH, BATCH)])
    pltpu.make_async_copy(adj_hbm.at[0, pl.ds(base, E_PER_T)], rowv, pr).wait()
    pltpu.make_async_copy(vals_hbm.at[pl.ds(base, E_PER_T)], valv, pv).wait()
    plsc.subcore_barrier()

    def scale(gj, b):
        def sb(j16, _):
            off = b * BATCH + j16 * LANES
            v16 = valv[pl.ds(off, LANES)]
            ebase = j16 * LANES
            for i in range(0, LANES, 2):
                va = jnp.full((LANES,), v16[i], jnp.float32)
                vb = jnp.full((LANES,), v16[i + 1], jnp.float32)
                ra = [gj[ebase + i, pl.ds(k * LANES, LANES)]
                      for k in range(NSEG)]
                rb = [gj[ebase + i + 1, pl.ds(k * LANES, LANES)]
                      for k in range(NSEG)]
                pa = [x * va for x in ra]
                pb = [x * vb for x in rb]
                for k in range(NSEG):
                    gj[ebase + i, pl.ds(k * LANES, LANES)] = pa[k]
                for k in range(NSEG):
                    gj[ebase + i + 1, pl.ds(k * LANES, LANES)] = pb[k]
            return 0

        lax.fori_loop(0, EPB16, sb, 0)

    def outer(a, _):
        for j in range(NBUF):
            b = a * NBUF + j
            j4 = (j + 4) % NBUF
            wait_gather(b, j)
            scale(g[j], b)
            for i in range(EPB16):
                r[j][pl.ds(i * LANES, LANES)] = \
                    rowv[pl.ds(b * BATCH + i * LANES, LANES)]
            pltpu.async_copy(g[j], acc.at[r[j]], ssem[j], add=True)

            @pl.when(b >= 1)
            def _():
                wait_scatter(j4)

            @pl.when(b + NBUF - 1 < NBATCH)
            def _():
                fire_gather(b + NBUF - 1, j4)
        return 0

    lax.fori_loop(0, NBATCH // NBUF, outer, 0)
    wait_scatter((NBATCH - 1) % NBUF)
    plsc.subcore_barrier()

    csl = pl.ds(h * HFEAT, HFEAT)

    @pl.when(s < NS - 1)
    def _():
        rsl = pl.ds(s * ROWS_PER_TILE, ROWS_PER_TILE)
        pltpu.sync_copy(acc.at[rsl], out.at[rsl, csl])

    @pl.when(s == NS - 1)
    def _():
        tail = N_NODES - (NS - 1) * ROWS_PER_TILE
        rslt = pl.ds((NS - 1) * ROWS_PER_TILE, tail)
        pltpu.sync_copy(acc.at[rslt], out.at[rslt, csl])


_sc_spmm_call = functools.partial(
    pl.kernel,
    out_type=jax.ShapeDtypeStruct((N_NODES, D_FEAT), jnp.float32),
    mesh=plsc.VectorSubcoreMesh(core_axis_name="c", subcore_axis_name="s"),
    compiler_params=pltpu.CompilerParams(use_tc_tiling_on_sc=False),
    scratch_types=(
        [pltpu.VMEM((BATCH, HFEAT), jnp.float32)] * NBUF
        + [pltpu.VMEM((BATCH,), jnp.int32)] * NBUF
        + [pltpu.VMEM((BATCH,), jnp.int32)] * NBUF
        + [
            pltpu.VMEM((E_PER_T,), jnp.int32),
            pltpu.VMEM((E_PER_T,), jnp.int32),
            pltpu.VMEM((E_PER_T,), jnp.float32),
            pltpu.VMEM_SHARED((N_PAD, HFEAT), jnp.float32),
        ]
        + [pltpu.SemaphoreType.DMA] * (2 * NBUF + 3)
    ),
)(_sc_spmm)


@jax.jit
def kernel(adj_indices, adj_values, embeds):
    adj = adj_indices.astype(jnp.int32)
    embeds2 = embeds.reshape(2 * N_NODES, HFEAT)
    return _sc_spmm_call(adj, adj_values, embeds2)


# speedup vs baseline: 2.7818x; 1.0024x over previous
import functools

import jax
import jax.numpy as jnp
from jax import lax
from jax.experimental import pallas as pl
from jax.experimental.pallas import tpu as pltpu
from jax.experimental.pallas import tpu_sc as plsc

N_NODES = 10000
N_EDGES = 320000
D_FEAT = 128

NC = 2
NS = 16
LANES = 16
HFEAT = D_FEAT // NC
E_PER_T = N_EDGES // NS
BATCH = 80
NBATCH = E_PER_T // BATCH
NBUF = 5
N_PAD = 10240
ROWS_PER_TILE = N_PAD // NS
NSEG = HFEAT // LANES
EPB16 = BATCH // LANES


def _sc_spmm(adj_hbm, vals_hbm, embeds_hbm,
             out,
             g0, g1, g2, g3, g4, r0, r1, r2, r3, r4,
             c0, c1, c2, c3, c4,
             colv, rowv, valv, acc,
             gs0, gs1, gs2, gs3, gs4, ss0, ss1, ss2, ss3, ss4,
             pc, pr, pv):
    g = [g0, g1, g2, g3, g4]
    r = [r0, r1, r2, r3, r4]
    cb = [c0, c1, c2, c3, c4]
    gsem = [gs0, gs1, gs2, gs3, gs4]
    ssem = [ss0, ss1, ss2, ss3, ss4]

    h = lax.axis_index("c")
    s = lax.axis_index("s")
    base = s * E_PER_T

    pltpu.async_copy(adj_hbm.at[1, pl.ds(base, E_PER_T)], colv, pc)
    pltpu.async_copy(adj_hbm.at[0, pl.ds(base, E_PER_T)], rowv, pr)
    pltpu.async_copy(vals_hbm.at[pl.ds(base, E_PER_T)], valv, pv)

    hvec = jnp.full((LANES,), h, jnp.int32)
    two = jnp.full((LANES,), 2, jnp.int32)

    def fire_gather(b, j):
        for i in range(EPB16):
            cb[j][pl.ds(i * LANES, LANES)] = \
                colv[pl.ds(b * BATCH + i * LANES, LANES)] * two + hvec
        pltpu.async_copy(embeds_hbm.at[cb[j]], g[j], gsem[j])

    def wait_gather(b, j):
        pltpu.make_async_copy(embeds_hbm.at[cb[j]], g[j], gsem[j]).wait()

    def wait_scatter(j):
        pltpu.make_async_copy(g[j], acc.at[r[j]], ssem[j]).wait()

    pltpu.make_async_copy(adj_hbm.at[1, pl.ds(base, E_PER_T)], colv, pc).wait()
    for j in range(NBUF - 1):
        fire_gather(j, j)

    zeros16 = jnp.zeros((LANES,), jnp.float32)

    def zero_body(i, _):
        for k in range(NSEG):
            g4[i, pl.ds(k * LANES, LANES)] = zeros16
        return 0

    lax.fori_loop(0, BATCH, zero_body, 0)
    for m in range(ROWS_PER_TILE // BATCH):
        pltpu.sync_copy(g4, acc.at[pl.ds(s * ROWS_PER_TILE + m * BATCH, BATCH)])
    pltpu.make_async_copy(adj_hbm.at[0, pl.ds(base, E_PER_T)], rowv, pr).wait()
    pltpu.make_async_copy(vals_hbm.at[pl.ds(base, E_PER_T)], valv, pv).wait()
    plsc.subcore_barrier()

    def scale(gj, b):
        def sb(j16, _):
            off = b * BATCH + j16 * LANES
            v16 = valv[pl.ds(off, LANES)]
            ebase = j16 * LANES
            for i in range(0, LANES, 4):
                vs = [jnp.full((LANES,), v16[i + e], jnp.float32)
                      for e in range(4)]
                rs = [[gj[ebase + i + e, pl.ds(k * LANES, LANES)]
                       for k in range(NSEG)] for e in range(4)]
                ps = [[x * vs[e] for x in rs[e]] for e in range(4)]
                for e in range(4):
                    for k in range(NSEG):
                        gj[ebase + i + e, pl.ds(k * LANES, LANES)] = ps[e][k]
            return 0

        lax.fori_loop(0, EPB16, sb, 0)

    def outer(a, _):
        for j in range(NBUF):
            b = a * NBUF + j
            j4 = (j + 4) % NBUF
            wait_gather(b, j)
            scale(g[j], b)
            for i in range(EPB16):
                r[j][pl.ds(i * LANES, LANES)] = \
                    rowv[pl.ds(b * BATCH + i * LANES, LANES)]
            pltpu.async_copy(g[j], acc.at[r[j]], ssem[j], add=True)

            @pl.when(b >= 1)
            def _():
                wait_scatter(j4)

            @pl.when(b + NBUF - 1 < NBATCH)
            def _():
                fire_gather(b + NBUF - 1, j4)
        return 0

    lax.fori_loop(0, NBATCH // NBUF, outer, 0)
    wait_scatter((NBATCH - 1) % NBUF)
    plsc.subcore_barrier()

    csl = pl.ds(h * HFEAT, HFEAT)

    @pl.when(s < NS - 1)
    def _():
        rsl = pl.ds(s * ROWS_PER_TILE, ROWS_PER_TILE)
        pltpu.sync_copy(acc.at[rsl], out.at[rsl, csl])

    @pl.when(s == NS - 1)
    def _():
        tail = N_NODES - (NS - 1) * ROWS_PER_TILE
        rslt = pl.ds((NS - 1) * ROWS_PER_TILE, tail)
        pltpu.sync_copy(acc.at[rslt], out.at[rslt, csl])


_sc_spmm_call = functools.partial(
    pl.kernel,
    out_type=jax.ShapeDtypeStruct((N_NODES, D_FEAT), jnp.float32),
    mesh=plsc.VectorSubcoreMesh(core_axis_name="c", subcore_axis_name="s"),
    compiler_params=pltpu.CompilerParams(use_tc_tiling_on_sc=False),
    scratch_types=(
        [pltpu.VMEM((BATCH, HFEAT), jnp.float32)] * NBUF
        + [pltpu.VMEM((BATCH,), jnp.int32)] * NBUF
        + [pltpu.VMEM((BATCH,), jnp.int32)] * NBUF
        + [
            pltpu.VMEM((E_PER_T,), jnp.int32),
            pltpu.VMEM((E_PER_T,), jnp.int32),
            pltpu.VMEM((E_PER_T,), jnp.float32),
            pltpu.VMEM_SHARED((N_PAD, HFEAT), jnp.float32),
        ]
        + [pltpu.SemaphoreType.DMA] * (2 * NBUF + 3)
    ),
)(_sc_spmm)


@jax.jit
def kernel(adj_indices, adj_values, embeds):
    adj = adj_indices.astype(jnp.int32)
    embeds2 = embeds.reshape(2 * N_NODES, HFEAT)
    return _sc_spmm_call(adj, adj_values, embeds2)
